# baseline (speedup 1.0000x reference)
import math

import jax
import jax.numpy as jnp
from jax import lax
from jax.experimental import pallas as pl
from jax.experimental.pallas import tpu as pltpu
from jax.experimental.pallas import tpu_sc as plsc

VOCAB = 32000
D_MODEL = 2048
HEAD_DIM = 128
MAX_LEN = 4096
THETA = 10000.0
B = 4
S = 4096

NEG_MIN = float(jnp.finfo(jnp.float32).min)


_NC = 2
_NS = 16
_NW = _NC * _NS
_N_TOK = B * S
_PER_W = _N_TOK // _NW
_CH = 16
_NCH = _PER_W // _CH
_NBUF = 2
_PREF = 2
_MROWS = S // _NW
_MPC = _MROWS // _NCH
_TW = 4224


def _sc_body(table_hbm, ids_hbm, tmpl_hbm, out_hbm, mask_hbm,
             idx_v, rows_v, tmpl_v, gsem, osem, msem):
    wid = lax.axis_index("s") * _NC + lax.axis_index("c")
    base = wid * _PER_W
    lo = pl.multiple_of((S - _MROWS) - wid * _MROWS, 128)

    pltpu.sync_copy(ids_hbm.at[wid], idx_v)

    def gather_copy(c):
        b = c % _NBUF
        return pltpu.make_async_copy(
            table_hbm.at[idx_v.at[c]], rows_v.at[b], gsem.at[b])

    def out_copy(c):
        b = c % _NBUF
        return pltpu.make_async_copy(
            rows_v.at[b], out_hbm.at[pl.ds(base + c * _CH, _CH)], osem.at[b])

    def mask_copy(m):
        p = 7 - lax.rem(m, 8)
        src = pl.multiple_of(p * _TW + (_MROWS - 8) - 8 * lax.div(m, 8), 8)
        i = wid * _MROWS + m
        dst = pl.multiple_of(i * S, 8)
        return pltpu.make_async_copy(
            tmpl_v.at[pl.ds(src, S)], mask_hbm.at[pl.ds(dst, S)], msem)

    for g in range(_PREF):
        gather_copy(g).start()

    for p in range(8):
        pltpu.sync_copy(
            tmpl_hbm.at[pl.ds(p * 2 * S + lo, _TW)],
            tmpl_v.at[pl.ds(p * _TW, _TW)])

    def body(c, carry):
        gather_copy(c).wait()
        oc = out_copy(c)
        oc.start()
        oc.wait()

        for k in range(_MPC):
            mask_copy(c * _MPC + k).start()

        @pl.when(c + _PREF < _NCH)
        def _():
            gather_copy(c + _PREF).start()

        return carry

    lax.fori_loop(0, _NCH, body, 0, unroll=False)

    def drain(m, carry):
        mask_copy(m).wait()
        return carry

    lax.fori_loop(0, _MROWS, drain, 0, unroll=False)


def _sc_gather_mask(table, ids3, tmpl):
    kern = pl.kernel(
        _sc_body,
        out_type=(
            jax.ShapeDtypeStruct((_N_TOK, D_MODEL), jnp.float32),
            jax.ShapeDtypeStruct((S * S,), jnp.float32),
        ),
        mesh=plsc.VectorSubcoreMesh(core_axis_name="c", subcore_axis_name="s"),
        scratch_types=[
            pltpu.VMEM((_NCH, _CH), jnp.int32),
            pltpu.VMEM((_NBUF, _CH, D_MODEL), jnp.float32),
            pltpu.VMEM((8 * _TW,), jnp.float32),
            pltpu.SemaphoreType.DMA((_NBUF,)),
            pltpu.SemaphoreType.DMA((_NBUF,)),
            pltpu.SemaphoreType.DMA,
        ],
    )
    return kern(table, ids3, tmpl)


_HD2 = HEAD_DIM // 2


def _tc_body(cos_ref, sin_ref, tmpl_ref):
    t = lax.broadcasted_iota(jnp.int32, (MAX_LEN, _HD2), 0).astype(jnp.float32)
    j = lax.broadcasted_iota(jnp.int32, (MAX_LEN, _HD2), 1).astype(jnp.float32)
    inv = jnp.exp(j * (-2.0 / HEAD_DIM * math.log(THETA)))
    f = t * inv
    cos_ref[...] = jnp.cos(f)
    sin_ref[...] = jnp.sin(f)
    pp = lax.broadcasted_iota(jnp.int32, (8, 2 * S), 0)
    qq = lax.broadcasted_iota(jnp.int32, (8, 2 * S), 1)
    tmpl_ref[...] = jnp.where(qq < S - pp, 0.0, NEG_MIN).astype(jnp.float32)


def _make_tc():
    return pl.pallas_call(
        _tc_body,
        out_shape=(
            jax.ShapeDtypeStruct((MAX_LEN, _HD2), jnp.float32),
            jax.ShapeDtypeStruct((MAX_LEN, _HD2), jnp.float32),
            jax.ShapeDtypeStruct((8, 2 * S), jnp.float32),
        ),
    )()


def kernel(input_ids, labels, table):
    ids3 = input_ids.reshape(_NW, _NCH, _CH).astype(jnp.int32)
    cos, sin, tmpl = _make_tc()
    hidden2d, mask1d = _sc_gather_mask(table, ids3, tmpl.reshape(8 * 2 * S))
    hidden = hidden2d.reshape(B, S, D_MODEL)
    mask = mask1d.reshape(1, 1, S, S)
    freqs_cis = lax.complex(cos, sin)
    return (hidden, freqs_cis, mask, labels)


# speedup vs baseline: 1.6309x; 1.6309x over previous
import math

import jax
import jax.numpy as jnp
from jax import lax
from jax.experimental import pallas as pl
from jax.experimental.pallas import tpu as pltpu
from jax.experimental.pallas import tpu_sc as plsc

VOCAB = 32000
D_MODEL = 2048
HEAD_DIM = 128
MAX_LEN = 4096
THETA = 10000.0
B = 4
S = 4096

NEG_MIN = float(jnp.finfo(jnp.float32).min)


_NC = 2
_NS = 16
_NW = _NC * _NS
_N_TOK = B * S
_PER_W = _N_TOK // _NW
_CH = 16
_NCH = _PER_W // _CH
_NBUF = 3
_PREF = 3


def _sc_gather_body(table_hbm, ids_hbm, out_hbm, idx_v, rows_v, gsem, osem):
    wid = lax.axis_index("s") * _NC + lax.axis_index("c")
    base = wid * _PER_W

    pltpu.sync_copy(ids_hbm.at[wid], idx_v)

    def gather_copy(c):
        b = c % _NBUF
        return pltpu.make_async_copy(
            table_hbm.at[idx_v.at[c]], rows_v.at[b], gsem.at[b])

    def out_copy(c):
        b = c % _NBUF
        return pltpu.make_async_copy(
            rows_v.at[b], out_hbm.at[pl.ds(base + c * _CH, _CH)], osem.at[b])

    for g in range(_PREF):
        gather_copy(g).start()

    def body(c, carry):
        gather_copy(c).wait()
        oc = out_copy(c)
        oc.start()
        oc.wait()

        @pl.when(c + _PREF < _NCH)
        def _():
            gather_copy(c + _PREF).start()

        return carry

    lax.fori_loop(0, _NCH, body, 0, unroll=False)


def _sc_gather(table, ids3):
    kern = pl.kernel(
        _sc_gather_body,
        out_type=jax.ShapeDtypeStruct((_N_TOK, D_MODEL), jnp.float32),
        mesh=plsc.VectorSubcoreMesh(core_axis_name="c", subcore_axis_name="s"),
        scratch_types=[
            pltpu.VMEM((_NCH, _CH), jnp.int32),
            pltpu.VMEM((_NBUF, _CH, D_MODEL), jnp.float32),
            pltpu.SemaphoreType.DMA((_NBUF,)),
            pltpu.SemaphoreType.DMA((_NBUF,)),
        ],
    )
    return kern(table, ids3)


_MBLK = 512
_HD2 = HEAD_DIM // 2


def _mask_freqs_body(mask_ref, cos_ref, sin_ref):
    i = pl.program_id(0)
    rows = lax.broadcasted_iota(jnp.int32, (_MBLK, S), 0) + i * _MBLK
    cols = lax.broadcasted_iota(jnp.int32, (_MBLK, S), 1)
    mask_ref[...] = jnp.where(cols > rows, NEG_MIN, 0.0).astype(jnp.float32)

    @pl.when(i == 0)
    def _():
        t = lax.broadcasted_iota(jnp.int32, (MAX_LEN, _HD2), 0).astype(jnp.float32)
        j = lax.broadcasted_iota(jnp.int32, (MAX_LEN, _HD2), 1).astype(jnp.float32)
        inv = jnp.exp(j * (-2.0 / HEAD_DIM * math.log(THETA)))
        f = t * inv
        cos_ref[...] = jnp.cos(f)
        sin_ref[...] = jnp.sin(f)


def _make_mask_freqs():
    return pl.pallas_call(
        _mask_freqs_body,
        grid=(S // _MBLK,),
        out_specs=[
            pl.BlockSpec((_MBLK, S), lambda i: (i, 0)),
            pl.BlockSpec((MAX_LEN, _HD2), lambda i: (0, 0)),
            pl.BlockSpec((MAX_LEN, _HD2), lambda i: (0, 0)),
        ],
        out_shape=[
            jax.ShapeDtypeStruct((S, S), jnp.float32),
            jax.ShapeDtypeStruct((MAX_LEN, _HD2), jnp.float32),
            jax.ShapeDtypeStruct((MAX_LEN, _HD2), jnp.float32),
        ],
    )()


def kernel(input_ids, labels, table):
    ids3 = input_ids.reshape(_NW, _NCH, _CH).astype(jnp.int32)
    hidden = _sc_gather(table, ids3).reshape(B, S, D_MODEL)
    mask2d, cos, sin = _make_mask_freqs()
    mask = mask2d.reshape(1, 1, S, S)
    freqs_cis = lax.complex(cos, sin)
    return (hidden, freqs_cis, mask, labels)


# speedup vs baseline: 1.6341x; 1.0019x over previous
import math

import jax
import jax.numpy as jnp
from jax import lax
from jax.experimental import pallas as pl
from jax.experimental.pallas import tpu as pltpu
from jax.experimental.pallas import tpu_sc as plsc

VOCAB = 32000
D_MODEL = 2048
HEAD_DIM = 128
MAX_LEN = 4096
THETA = 10000.0
B = 4
S = 4096

NEG_MIN = float(jnp.finfo(jnp.float32).min)


_NC = 2
_NS = 16
_NW = _NC * _NS
_N_TOK = B * S
_PER_W = _N_TOK // _NW
_CH = 16
_NCH = _PER_W // _CH
_NBUF = 3
_PREF = 3


def _sc_gather_body(table_hbm, ids_hbm, out_hbm, idx_v, rows_v, gsem, osem):
    wid = lax.axis_index("s") * _NC + lax.axis_index("c")
    base = wid * _PER_W

    pltpu.sync_copy(ids_hbm.at[wid], idx_v)

    def gather_copy(c):
        b = c % _NBUF
        return pltpu.make_async_copy(
            table_hbm.at[idx_v.at[c]], rows_v.at[b], gsem.at[b])

    def out_copy(c):
        b = c % _NBUF
        return pltpu.make_async_copy(
            rows_v.at[b], out_hbm.at[pl.ds(base + c * _CH, _CH)], osem.at[b])

    for g in range(_PREF):
        gather_copy(g).start()

    def body(c, carry):
        gather_copy(c).wait()
        oc = out_copy(c)
        oc.start()
        oc.wait()

        @pl.when(c + _PREF < _NCH)
        def _():
            gather_copy(c + _PREF).start()

        return carry

    lax.fori_loop(0, _NCH, body, 0, unroll=False)


def _sc_gather(table, ids3):
    kern = pl.kernel(
        _sc_gather_body,
        out_type=jax.ShapeDtypeStruct((_N_TOK, D_MODEL), jnp.float32),
        mesh=plsc.VectorSubcoreMesh(core_axis_name="c", subcore_axis_name="s"),
        scratch_types=[
            pltpu.VMEM((_NCH, _CH), jnp.int32),
            pltpu.VMEM((_NBUF, _CH, D_MODEL), jnp.float32),
            pltpu.SemaphoreType.DMA((_NBUF,)),
            pltpu.SemaphoreType.DMA((_NBUF,)),
        ],
    )
    return kern(table, ids3)


_MBLK = 512
_HD2 = HEAD_DIM // 2


def _mask_body(o_ref):
    i = pl.program_id(0)
    rows = lax.broadcasted_iota(jnp.int32, (_MBLK, S), 0) + i * _MBLK
    cols = lax.broadcasted_iota(jnp.int32, (_MBLK, S), 1)
    o_ref[...] = jnp.where(cols > rows, NEG_MIN, 0.0).astype(jnp.float32)


def _make_mask():
    return pl.pallas_call(
        _mask_body,
        grid=(S // _MBLK,),
        out_specs=pl.BlockSpec((_MBLK, S), lambda i: (i, 0)),
        out_shape=jax.ShapeDtypeStruct((S, S), jnp.float32),
    )()


def _freqs_body(cos_ref, sin_ref):
    t = lax.broadcasted_iota(jnp.int32, (MAX_LEN, _HD2), 0).astype(jnp.float32)
    j = lax.broadcasted_iota(jnp.int32, (MAX_LEN, _HD2), 1).astype(jnp.float32)
    inv = jnp.exp(j * (-2.0 / HEAD_DIM * math.log(THETA)))
    f = t * inv
    cos_ref[...] = jnp.cos(f)
    sin_ref[...] = jnp.sin(f)


def _make_freqs():
    return pl.pallas_call(
        _freqs_body,
        out_shape=(
            jax.ShapeDtypeStruct((MAX_LEN, _HD2), jnp.float32),
            jax.ShapeDtypeStruct((MAX_LEN, _HD2), jnp.float32),
        ),
    )()


def kernel(input_ids, labels, table):
    ids3 = input_ids.reshape(_NW, _NCH, _CH).astype(jnp.int32)
    hidden = _sc_gather(table, ids3).reshape(B, S, D_MODEL)
    mask = _make_mask().reshape(1, 1, S, S)
    cos, sin = _make_freqs()
    freqs_cis = lax.complex(cos, sin)
    return (hidden, freqs_cis, mask, labels)
